# baseline (speedup 1.0000x reference)
import functools

import jax
import jax.numpy as jnp
from jax import lax
from jax.experimental import pallas as pl
from jax.experimental.pallas import tpu as pltpu
from jax.experimental.pallas import tpu_sc as plsc

DT = 0.03

_NC = 2
_NS = 16
_NW = _NC * _NS


def _gather_body(h_hbm, row_hbm, col_hbm, hcat_hbm,
                 ridx, cidx, ra0, ca0, ra1, ca1,
                 gsa, gsb, wsa, wsb, *, epw, ch, d):
    c = lax.axis_index("c")
    s = lax.axis_index("s")
    wid = s * _NC + c
    base = wid * epw
    npairs = epw // ch

    pltpu.sync_copy(row_hbm.at[pl.ds(base, epw)], ridx)
    pltpu.sync_copy(col_hbm.at[pl.ds(base, epw)], cidx)

    def fire_gather(p, bufs, sem):
        ra, ca = bufs
        gr = pltpu.async_copy(h_hbm.at[ridx.at[pl.ds(p * ch, ch)]], ra, sem)
        gc = pltpu.async_copy(h_hbm.at[cidx.at[pl.ds(p * ch, ch)]], ca, sem)
        return gr, gc

    def drain_gather(p, bufs, sem):
        ra, ca = bufs
        pltpu.make_async_copy(h_hbm.at[ridx.at[pl.ds(p * ch, ch)]], ra,
                              sem).wait()
        pltpu.make_async_copy(h_hbm.at[cidx.at[pl.ds(p * ch, ch)]], ca,
                              sem).wait()

    def fire_write(p, bufs, sem):
        ra, ca = bufs
        off = base + p * ch
        pltpu.async_copy(ra, hcat_hbm.at[pl.ds(off, ch), pl.ds(0, d)], sem)
        pltpu.async_copy(ca, hcat_hbm.at[pl.ds(off, ch), pl.ds(d, d)], sem)

    def drain_write(p, bufs, sem):
        ra, ca = bufs
        off = base + p * ch
        pltpu.make_async_copy(ra, hcat_hbm.at[pl.ds(off, ch), pl.ds(0, d)],
                              sem).wait()
        pltpu.make_async_copy(ca, hcat_hbm.at[pl.ds(off, ch), pl.ds(d, d)],
                              sem).wait()

    bufs0 = (ra0, ca0)
    bufs1 = (ra1, ca1)
    fire_gather(0, bufs0, gsa)
    fire_gather(1, bufs1, gsb)
    drain_gather(0, bufs0, gsa)
    fire_write(0, bufs0, wsa)

    def body(i, _):
        pa = 2 + 2 * i
        pb = 3 + 2 * i
        drain_write(pa - 2, bufs0, wsa)
        fire_gather(pa, bufs0, gsa)
        drain_gather(pa - 1, bufs1, gsb)
        fire_write(pa - 1, bufs1, wsb)
        drain_write(pb - 2, bufs1, wsb)
        fire_gather(pb, bufs1, gsb)
        drain_gather(pb - 1, bufs0, gsa)
        fire_write(pb - 1, bufs0, wsa)
        return 0

    lax.fori_loop(0, (npairs - 2) // 2, body, 0)
    last = npairs - 1
    drain_gather(last, bufs1, gsb)
    fire_write(last, bufs1, wsb)
    drain_write(last - 1, bufs0, wsa)
    drain_write(last, bufs1, wsb)


def _sc_gather(h, row1, col1, ch):
    e = row1.shape[0]
    d = h.shape[1]
    epw = e // _NW
    mesh = plsc.VectorSubcoreMesh(core_axis_name="c", subcore_axis_name="s")
    kern = pl.kernel(
        functools.partial(_gather_body, epw=epw, ch=ch, d=d),
        out_type=jax.ShapeDtypeStruct((e, 2 * d), h.dtype),
        mesh=mesh,
        scratch_types=[
            pltpu.VMEM((epw,), jnp.int32),
            pltpu.VMEM((epw,), jnp.int32),
            pltpu.VMEM((ch, d), h.dtype),
            pltpu.VMEM((ch, d), h.dtype),
            pltpu.VMEM((ch, d), h.dtype),
            pltpu.VMEM((ch, d), h.dtype),
            pltpu.SemaphoreType.DMA,
            pltpu.SemaphoreType.DMA,
            pltpu.SemaphoreType.DMA,
            pltpu.SemaphoreType.DMA,
        ],
    )
    return kern(h, row1, col1)


def _scatter_body(*refs, epc, ch, nps, nchunks):
    msgs = refs[:nchunks]
    row_hbm, zeros_hbm, out_hbm, idx_v, msg_v, shared, sem = refs[nchunks:]
    c = lax.axis_index("c")
    s = lax.axis_index("s")
    wid = s * _NC + c

    pltpu.sync_copy(zeros_hbm, shared.at[pl.ds(s * nps, nps)])
    plsc.subcore_barrier()

    for q, mref in enumerate(msgs):
        base = wid * epc

        def body(i, _):
            off = base + i * ch
            d0 = pltpu.async_copy(row_hbm.at[pl.ds(q * epc * _NW + off, ch)],
                                  idx_v, sem)
            d1 = pltpu.async_copy(mref.at[pl.ds(off, ch)], msg_v, sem)
            d0.wait(); d1.wait()
            pltpu.sync_copy(msg_v, shared.at[idx_v], add=True)
            return 0

        lax.fori_loop(0, epc // ch, body, 0)

    plsc.subcore_barrier()

    npad = nps * _NS
    pltpu.sync_copy(shared.at[pl.ds(s * nps, nps)],
                    out_hbm.at[pl.ds(c * npad + s * nps, nps)])


def _sc_scatter(msgs, row, n_pad):
    ec, d = msgs[0].shape
    epc = ec // _NW
    ch = 200
    nps = n_pad // _NS
    zeros = jnp.zeros((nps, d), msgs[0].dtype)
    mesh = plsc.VectorSubcoreMesh(core_axis_name="c", subcore_axis_name="s")
    kern = pl.kernel(
        functools.partial(_scatter_body, epc=epc, ch=ch, nps=nps,
                          nchunks=len(msgs)),
        out_type=jax.ShapeDtypeStruct((_NC * n_pad, d), msgs[0].dtype),
        mesh=mesh,
        scratch_types=[
            pltpu.VMEM((ch,), jnp.int32),
            pltpu.VMEM((ch, d), msgs[0].dtype),
            pltpu.VMEM_SHARED((n_pad, d), msgs[0].dtype),
            pltpu.SemaphoreType.DMA,
        ],
    )
    return kern(*msgs, row, zeros)


def _mlp_body(x_ref, w1, b1, vw2t, vb2, w2f, w2p, c2, out_ref):
    f32 = jnp.float32
    bf = jnp.bfloat16
    d = x_ref.shape[1] // 2
    x = x_ref[...]
    xb = x.astype(bf)
    z = jnp.dot(xb, w1[...], preferred_element_type=f32) + b1[...]
    tv = jnp.tanh(z[:, :d])
    tf = jax.nn.relu(z[:, d:2 * d])
    tp = jnp.tanh(z[:, 2 * d:])
    s = (jnp.dot(tf.astype(bf), w2f[...], preferred_element_type=f32)
         + jnp.dot(tp.astype(bf), w2p[...], preferred_element_type=f32)
         + c2[...])
    nu = jnp.dot(tv.astype(bf), vw2t[...], preferred_element_type=f32) + vb2[...]
    out_ref[...] = s + nu * (x[:, d:] - x[:, :d])


def _tc_mlp(hcat, weights, d):
    e = hcat.shape[0]
    be = 2000
    grid = e // be
    in_spec = pl.BlockSpec((be, 2 * d), lambda i: (i, 0))
    out_spec = pl.BlockSpec((be, d), lambda i: (i, 0))
    full = lambda a: pl.BlockSpec(a.shape, lambda i: tuple(0 for _ in a.shape))
    return pl.pallas_call(
        _mlp_body,
        out_shape=jax.ShapeDtypeStruct((e, d), jnp.float32),
        grid=(grid,),
        in_specs=[in_spec] + [full(w) for w in weights],
        out_specs=out_spec,
    )(hcat, *weights)


def _combine_body(h_ref, p0_ref, p1_ref, out_ref):
    out_ref[...] = h_ref[...] + DT * (p0_ref[...] + p1_ref[...])


def _tc_combine(h, partials, n_pad):
    n, d = h.shape
    bn = 80
    spec = pl.BlockSpec((bn, d), lambda i: (i, 0))
    p1_spec = pl.BlockSpec((bn, d), lambda i: (i + n_pad // bn, 0))
    return pl.pallas_call(
        _combine_body,
        out_shape=jax.ShapeDtypeStruct((n, d), jnp.float32),
        grid=(n // bn,),
        in_specs=[spec, spec, p1_spec],
        out_specs=spec,
    )(h, partials, partials)


def kernel(h, edge_index, vw1, vb1, vw2, vb2, pw1, pb1, pw2, pb2,
           fw1, fb1, fw2, fb2):
    n, d = h.shape
    f32 = jnp.float32
    bf = jnp.bfloat16
    row = edge_index[0]
    col = edge_index[1]

    w1 = jnp.concatenate([
        jnp.concatenate([vw1[:d], fw1[:d], pw1], axis=1),
        jnp.concatenate([vw1[d:], fw1[d:], -pw1], axis=1),
    ], axis=0).astype(bf)
    b1 = jnp.concatenate([vb1, fb1, pb1]).reshape(1, 3 * d).astype(f32)
    weights = (
        w1, b1,
        jnp.tile(vw2, (1, d)).astype(bf),
        vb2.reshape(1, 1).astype(f32),
        fw2.astype(bf), (-pw2).astype(bf),
        (fb2 - pb2).reshape(1, d).astype(f32),
    )
    e = row.shape[0]
    nchunks = 5
    ec = e // nchunks
    msgs = []
    ch = 200
    for k in range(nchunks):
        sl = slice(k * ec, (k + 1) * ec)
        hcat_k = _sc_gather(h, row[sl], col[sl], ch)
        msgs.append(_tc_mlp(hcat_k, weights, d))

    n_pad = ((n + _NW * 8 - 1) // (_NW * 8)) * (_NW * 8)
    partials = _sc_scatter(msgs, row, n_pad)

    return _tc_combine(h, partials, n_pad)


# speedup vs baseline: 1.0073x; 1.0073x over previous
import functools

import jax
import jax.numpy as jnp
from jax import lax
from jax.experimental import pallas as pl
from jax.experimental.pallas import tpu as pltpu
from jax.experimental.pallas import tpu_sc as plsc

DT = 0.03

_NC = 2
_NS = 16
_NW = _NC * _NS


def _gather_body(h_hbm, row_hbm, col_hbm, hcat_hbm,
                 ridx, cidx, ra0, ca0, ra1, ca1,
                 gsa, gsb, wsa, wsb, *, epw, ch, d):
    c = lax.axis_index("c")
    s = lax.axis_index("s")
    wid = s * _NC + c
    base = wid * epw
    npairs = epw // ch

    pltpu.sync_copy(row_hbm.at[pl.ds(base, epw)], ridx)
    pltpu.sync_copy(col_hbm.at[pl.ds(base, epw)], cidx)

    def fire_gather(p, bufs, sem):
        ra, ca = bufs
        gr = pltpu.async_copy(h_hbm.at[ridx.at[pl.ds(p * ch, ch)]], ra, sem)
        gc = pltpu.async_copy(h_hbm.at[cidx.at[pl.ds(p * ch, ch)]], ca, sem)
        return gr, gc

    def drain_gather(p, bufs, sem):
        ra, ca = bufs
        pltpu.make_async_copy(h_hbm.at[ridx.at[pl.ds(p * ch, ch)]], ra,
                              sem).wait()
        pltpu.make_async_copy(h_hbm.at[cidx.at[pl.ds(p * ch, ch)]], ca,
                              sem).wait()

    def fire_write(p, bufs, sem):
        ra, ca = bufs
        off = base + p * ch
        pltpu.async_copy(ra, hcat_hbm.at[pl.ds(off, ch), pl.ds(0, d)], sem)
        pltpu.async_copy(ca, hcat_hbm.at[pl.ds(off, ch), pl.ds(d, d)], sem)

    def drain_write(p, bufs, sem):
        ra, ca = bufs
        off = base + p * ch
        pltpu.make_async_copy(ra, hcat_hbm.at[pl.ds(off, ch), pl.ds(0, d)],
                              sem).wait()
        pltpu.make_async_copy(ca, hcat_hbm.at[pl.ds(off, ch), pl.ds(d, d)],
                              sem).wait()

    bufs0 = (ra0, ca0)
    bufs1 = (ra1, ca1)

    def body(i, _):
        pa = 2 * i
        pb = 2 * i + 1
        fire_gather(pa, bufs0, gsa)
        fire_gather(pb, bufs1, gsb)
        drain_gather(pa, bufs0, gsa)
        drain_gather(pb, bufs1, gsb)
        fire_write(pa, bufs0, wsa)
        fire_write(pb, bufs1, wsb)
        drain_write(pa, bufs0, wsa)
        drain_write(pb, bufs1, wsb)
        return 0

    lax.fori_loop(0, npairs // 2, body, 0)


def _sc_gather(h, row1, col1, ch):
    e = row1.shape[0]
    d = h.shape[1]
    epw = e // _NW
    mesh = plsc.VectorSubcoreMesh(core_axis_name="c", subcore_axis_name="s")
    kern = pl.kernel(
        functools.partial(_gather_body, epw=epw, ch=ch, d=d),
        out_type=jax.ShapeDtypeStruct((e, 2 * d), h.dtype),
        mesh=mesh,
        scratch_types=[
            pltpu.VMEM((epw,), jnp.int32),
            pltpu.VMEM((epw,), jnp.int32),
            pltpu.VMEM((ch, d), h.dtype),
            pltpu.VMEM((ch, d), h.dtype),
            pltpu.VMEM((ch, d), h.dtype),
            pltpu.VMEM((ch, d), h.dtype),
            pltpu.SemaphoreType.DMA,
            pltpu.SemaphoreType.DMA,
            pltpu.SemaphoreType.DMA,
            pltpu.SemaphoreType.DMA,
        ],
    )
    return kern(h, row1, col1)


def _scatter_body(*refs, epc, ch, nps, nchunks):
    msgs = refs[:nchunks]
    row_hbm, zeros_hbm, out_hbm, idx_v, msg_v, shared, sem = refs[nchunks:]
    c = lax.axis_index("c")
    s = lax.axis_index("s")
    wid = s * _NC + c

    pltpu.sync_copy(zeros_hbm, shared.at[pl.ds(s * nps, nps)])
    plsc.subcore_barrier()

    for q, mref in enumerate(msgs):
        base = wid * epc

        def body(i, _):
            off = base + i * ch
            d0 = pltpu.async_copy(row_hbm.at[pl.ds(q * epc * _NW + off, ch)],
                                  idx_v, sem)
            d1 = pltpu.async_copy(mref.at[pl.ds(off, ch)], msg_v, sem)
            d0.wait(); d1.wait()
            pltpu.sync_copy(msg_v, shared.at[idx_v], add=True)
            return 0

        lax.fori_loop(0, epc // ch, body, 0)

    plsc.subcore_barrier()

    npad = nps * _NS
    pltpu.sync_copy(shared.at[pl.ds(s * nps, nps)],
                    out_hbm.at[pl.ds(c * npad + s * nps, nps)])


def _sc_scatter(msgs, row, n_pad):
    ec, d = msgs[0].shape
    epc = ec // _NW
    ch = 200
    nps = n_pad // _NS
    zeros = jnp.zeros((nps, d), msgs[0].dtype)
    mesh = plsc.VectorSubcoreMesh(core_axis_name="c", subcore_axis_name="s")
    kern = pl.kernel(
        functools.partial(_scatter_body, epc=epc, ch=ch, nps=nps,
                          nchunks=len(msgs)),
        out_type=jax.ShapeDtypeStruct((_NC * n_pad, d), msgs[0].dtype),
        mesh=mesh,
        scratch_types=[
            pltpu.VMEM((ch,), jnp.int32),
            pltpu.VMEM((ch, d), msgs[0].dtype),
            pltpu.VMEM_SHARED((n_pad, d), msgs[0].dtype),
            pltpu.SemaphoreType.DMA,
        ],
    )
    return kern(*msgs, row, zeros)


def _mlp_body(x_ref, w1, b1, vw2t, vb2, w2f, w2p, c2, out_ref):
    f32 = jnp.float32
    bf = jnp.bfloat16
    d = x_ref.shape[1] // 2
    x = x_ref[...]
    xb = x.astype(bf)
    z = jnp.dot(xb, w1[...], preferred_element_type=f32) + b1[...]
    tv = jnp.tanh(z[:, :d])
    tf = jax.nn.relu(z[:, d:2 * d])
    tp = jnp.tanh(z[:, 2 * d:])
    s = (jnp.dot(tf.astype(bf), w2f[...], preferred_element_type=f32)
         + jnp.dot(tp.astype(bf), w2p[...], preferred_element_type=f32)
         + c2[...])
    nu = jnp.dot(tv.astype(bf), vw2t[...], preferred_element_type=f32) + vb2[...]
    out_ref[...] = s + nu * (x[:, d:] - x[:, :d])


def _tc_mlp(hcat, weights, d):
    e = hcat.shape[0]
    be = 2000
    grid = e // be
    in_spec = pl.BlockSpec((be, 2 * d), lambda i: (i, 0))
    out_spec = pl.BlockSpec((be, d), lambda i: (i, 0))
    full = lambda a: pl.BlockSpec(a.shape, lambda i: tuple(0 for _ in a.shape))
    return pl.pallas_call(
        _mlp_body,
        out_shape=jax.ShapeDtypeStruct((e, d), jnp.float32),
        grid=(grid,),
        in_specs=[in_spec] + [full(w) for w in weights],
        out_specs=out_spec,
    )(hcat, *weights)


def _combine_body(h_ref, p0_ref, p1_ref, out_ref):
    out_ref[...] = h_ref[...] + DT * (p0_ref[...] + p1_ref[...])


def _tc_combine(h, partials, n_pad):
    n, d = h.shape
    bn = 80
    spec = pl.BlockSpec((bn, d), lambda i: (i, 0))
    p1_spec = pl.BlockSpec((bn, d), lambda i: (i + n_pad // bn, 0))
    return pl.pallas_call(
        _combine_body,
        out_shape=jax.ShapeDtypeStruct((n, d), jnp.float32),
        grid=(n // bn,),
        in_specs=[spec, spec, p1_spec],
        out_specs=spec,
    )(h, partials, partials)


def kernel(h, edge_index, vw1, vb1, vw2, vb2, pw1, pb1, pw2, pb2,
           fw1, fb1, fw2, fb2):
    n, d = h.shape
    f32 = jnp.float32
    bf = jnp.bfloat16
    row = edge_index[0]
    col = edge_index[1]

    w1 = jnp.concatenate([
        jnp.concatenate([vw1[:d], fw1[:d], pw1], axis=1),
        jnp.concatenate([vw1[d:], fw1[d:], -pw1], axis=1),
    ], axis=0).astype(bf)
    b1 = jnp.concatenate([vb1, fb1, pb1]).reshape(1, 3 * d).astype(f32)
    weights = (
        w1, b1,
        jnp.tile(vw2, (1, d)).astype(bf),
        vb2.reshape(1, 1).astype(f32),
        fw2.astype(bf), (-pw2).astype(bf),
        (fb2 - pb2).reshape(1, d).astype(f32),
    )
    e = row.shape[0]
    nchunks = 5
    ec = e // nchunks
    msgs = []
    ch = 200
    for k in range(nchunks):
        sl = slice(k * ec, (k + 1) * ec)
        hcat_k = _sc_gather(h, row[sl], col[sl], ch)
        msgs.append(_tc_mlp(hcat_k, weights, d))

    n_pad = ((n + _NW * 8 - 1) // (_NW * 8)) * (_NW * 8)
    partials = _sc_scatter(msgs, row, n_pad)

    return _tc_combine(h, partials, n_pad)


# speedup vs baseline: 1.1019x; 1.0939x over previous
import functools

import jax
import jax.numpy as jnp
from jax import lax
from jax.experimental import pallas as pl
from jax.experimental.pallas import tpu as pltpu
from jax.experimental.pallas import tpu_sc as plsc

DT = 0.03

_NC = 2
_NS = 16
_NW = _NC * _NS


def _gather_body(h_hbm, row_hbm, col_hbm, hcat_hbm,
                 ridx, cidx, ra0, ca0, ra1, ca1,
                 gsa, gsb, wsa, wsb, *, epw, ch, d):
    c = lax.axis_index("c")
    s = lax.axis_index("s")
    wid = s * _NC + c
    base = wid * epw
    npairs = epw // ch

    pltpu.sync_copy(row_hbm.at[pl.ds(base, epw)], ridx)
    pltpu.sync_copy(col_hbm.at[pl.ds(base, epw)], cidx)

    def fire_gather(p, bufs, sem):
        ra, ca = bufs
        gr = pltpu.async_copy(h_hbm.at[ridx.at[pl.ds(p * ch, ch)]], ra, sem)
        gc = pltpu.async_copy(h_hbm.at[cidx.at[pl.ds(p * ch, ch)]], ca, sem)
        return gr, gc

    def drain_gather(p, bufs, sem):
        ra, ca = bufs
        pltpu.make_async_copy(h_hbm.at[ridx.at[pl.ds(p * ch, ch)]], ra,
                              sem).wait()
        pltpu.make_async_copy(h_hbm.at[cidx.at[pl.ds(p * ch, ch)]], ca,
                              sem).wait()

    def fire_write(p, bufs, sem):
        ra, ca = bufs
        off = base + p * ch
        pltpu.async_copy(ra, hcat_hbm.at[pl.ds(off, ch), pl.ds(0, d)], sem)
        pltpu.async_copy(ca, hcat_hbm.at[pl.ds(off, ch), pl.ds(d, d)], sem)

    def drain_write(p, bufs, sem):
        ra, ca = bufs
        off = base + p * ch
        pltpu.make_async_copy(ra, hcat_hbm.at[pl.ds(off, ch), pl.ds(0, d)],
                              sem).wait()
        pltpu.make_async_copy(ca, hcat_hbm.at[pl.ds(off, ch), pl.ds(d, d)],
                              sem).wait()

    bufs0 = (ra0, ca0)
    bufs1 = (ra1, ca1)

    def body(i, _):
        pa = 2 * i
        pb = 2 * i + 1
        fire_gather(pa, bufs0, gsa)
        fire_gather(pb, bufs1, gsb)
        drain_gather(pa, bufs0, gsa)
        drain_gather(pb, bufs1, gsb)
        fire_write(pa, bufs0, wsa)
        fire_write(pb, bufs1, wsb)
        drain_write(pa, bufs0, wsa)
        drain_write(pb, bufs1, wsb)
        return 0

    lax.fori_loop(0, npairs // 2, body, 0)


def _sc_gather(h, row1, col1, ch):
    e = row1.shape[0]
    d = h.shape[1]
    epw = e // _NW
    mesh = plsc.VectorSubcoreMesh(core_axis_name="c", subcore_axis_name="s")
    kern = pl.kernel(
        functools.partial(_gather_body, epw=epw, ch=ch, d=d),
        out_type=jax.ShapeDtypeStruct((e, 2 * d), h.dtype),
        mesh=mesh,
        scratch_types=[
            pltpu.VMEM((epw,), jnp.int32),
            pltpu.VMEM((epw,), jnp.int32),
            pltpu.VMEM((ch, d), h.dtype),
            pltpu.VMEM((ch, d), h.dtype),
            pltpu.VMEM((ch, d), h.dtype),
            pltpu.VMEM((ch, d), h.dtype),
            pltpu.SemaphoreType.DMA,
            pltpu.SemaphoreType.DMA,
            pltpu.SemaphoreType.DMA,
            pltpu.SemaphoreType.DMA,
        ],
    )
    return kern(h, row1, col1)


def _scatter_body(*refs, epc, ch, nps, nchunks, chunk0):
    msgs = refs[:nchunks]
    row_hbm, zeros_hbm, out_hbm, idx_v, msg_v, shared, sem = refs[nchunks:]
    c = lax.axis_index("c")
    s = lax.axis_index("s")
    wid = s * _NC + c

    pltpu.sync_copy(zeros_hbm, shared.at[pl.ds(s * nps, nps)])
    plsc.subcore_barrier()

    for q0, mref in enumerate(msgs):
        q = q0 + chunk0
        base = wid * epc

        def body(i, _):
            off = base + i * ch
            d0 = pltpu.async_copy(row_hbm.at[pl.ds(q * epc * _NW + off, ch)],
                                  idx_v, sem)
            d1 = pltpu.async_copy(mref.at[pl.ds(off, ch)], msg_v, sem)
            d0.wait(); d1.wait()
            pltpu.sync_copy(msg_v, shared.at[idx_v], add=True)
            return 0

        lax.fori_loop(0, epc // ch, body, 0)

    plsc.subcore_barrier()

    npad = nps * _NS
    pltpu.sync_copy(shared.at[pl.ds(s * nps, nps)],
                    out_hbm.at[pl.ds(c * npad + s * nps, nps)])


def _sc_scatter(msgs, row, n_pad, chunk0):
    ec, d = msgs[0].shape
    epc = ec // _NW
    ch = 200
    nps = n_pad // _NS
    zeros = jnp.zeros((nps, d), msgs[0].dtype)
    mesh = plsc.VectorSubcoreMesh(core_axis_name="c", subcore_axis_name="s")
    kern = pl.kernel(
        functools.partial(_scatter_body, epc=epc, ch=ch, nps=nps,
                          nchunks=len(msgs), chunk0=chunk0),
        out_type=jax.ShapeDtypeStruct((_NC * n_pad, d), msgs[0].dtype),
        mesh=mesh,
        scratch_types=[
            pltpu.VMEM((ch,), jnp.int32),
            pltpu.VMEM((ch, d), msgs[0].dtype),
            pltpu.VMEM_SHARED((n_pad, d), msgs[0].dtype),
            pltpu.SemaphoreType.DMA,
        ],
    )
    return kern(*msgs, row, zeros)


def _mlp_body(x_ref, w1, b1, vw2t, vb2, w2f, w2p, c2, out_ref):
    f32 = jnp.float32
    bf = jnp.bfloat16
    d = x_ref.shape[1] // 2
    x = x_ref[...]
    xb = x.astype(bf)
    z = jnp.dot(xb, w1[...], preferred_element_type=f32) + b1[...]
    tv = jnp.tanh(z[:, :d])
    tf = jax.nn.relu(z[:, d:2 * d])
    tp = jnp.tanh(z[:, 2 * d:])
    s = (jnp.dot(tf.astype(bf), w2f[...], preferred_element_type=f32)
         + jnp.dot(tp.astype(bf), w2p[...], preferred_element_type=f32)
         + c2[...])
    nu = jnp.dot(tv.astype(bf), vw2t[...], preferred_element_type=f32) + vb2[...]
    out_ref[...] = s + nu * (x[:, d:] - x[:, :d])


def _tc_mlp(hcat, weights, d):
    e = hcat.shape[0]
    be = 2000
    grid = e // be
    in_spec = pl.BlockSpec((be, 2 * d), lambda i: (i, 0))
    out_spec = pl.BlockSpec((be, d), lambda i: (i, 0))
    full = lambda a: pl.BlockSpec(a.shape, lambda i: tuple(0 for _ in a.shape))
    return pl.pallas_call(
        _mlp_body,
        out_shape=jax.ShapeDtypeStruct((e, d), jnp.float32),
        grid=(grid,),
        in_specs=[in_spec] + [full(w) for w in weights],
        out_specs=out_spec,
    )(hcat, *weights)


def _combine_body(h_ref, a0_ref, a1_ref, b0_ref, b1_ref, out_ref):
    out_ref[...] = h_ref[...] + DT * ((a0_ref[...] + a1_ref[...])
                                      + (b0_ref[...] + b1_ref[...]))


def _tc_combine(h, pa, pb, n_pad):
    n, d = h.shape
    bn = 80
    spec = pl.BlockSpec((bn, d), lambda i: (i, 0))
    p1_spec = pl.BlockSpec((bn, d), lambda i: (i + n_pad // bn, 0))
    return pl.pallas_call(
        _combine_body,
        out_shape=jax.ShapeDtypeStruct((n, d), jnp.float32),
        grid=(n // bn,),
        in_specs=[spec, spec, p1_spec, spec, p1_spec],
        out_specs=spec,
    )(h, pa, pa, pb, pb)


def kernel(h, edge_index, vw1, vb1, vw2, vb2, pw1, pb1, pw2, pb2,
           fw1, fb1, fw2, fb2):
    n, d = h.shape
    f32 = jnp.float32
    bf = jnp.bfloat16
    row = edge_index[0]
    col = edge_index[1]

    w1 = jnp.concatenate([
        jnp.concatenate([vw1[:d], fw1[:d], pw1], axis=1),
        jnp.concatenate([vw1[d:], fw1[d:], -pw1], axis=1),
    ], axis=0).astype(bf)
    b1 = jnp.concatenate([vb1, fb1, pb1]).reshape(1, 3 * d).astype(f32)
    weights = (
        w1, b1,
        jnp.tile(vw2, (1, d)).astype(bf),
        vb2.reshape(1, 1).astype(f32),
        fw2.astype(bf), (-pw2).astype(bf),
        (fb2 - pb2).reshape(1, d).astype(f32),
    )
    e = row.shape[0]
    nchunks = 5
    ec = e // nchunks
    msgs = []
    ch = 200
    for k in range(nchunks):
        sl = slice(k * ec, (k + 1) * ec)
        hcat_k = _sc_gather(h, row[sl], col[sl], ch)
        msgs.append(_tc_mlp(hcat_k, weights, d))

    n_pad = ((n + _NW * 8 - 1) // (_NW * 8)) * (_NW * 8)
    pa = _sc_scatter(msgs[:3], row, n_pad, 0)
    pb = _sc_scatter(msgs[3:], row, n_pad, 3)

    return _tc_combine(h, pa, pb, n_pad)
